# baseline (device time: 18475 ns/iter reference)
import jax
import jax.numpy as jnp
from jax import lax
from jax.experimental import pallas as pl
from jax.experimental.pallas import tpu as pltpu

N_DEV = 16


def kernel(x, dy, gamma):
    m, d = x.shape

    def body(x_ref, dy_ref, out_ref, own_ref, comm_ref,
             send_sems, recv_sems, exit_sems):
        my = lax.axis_index("i")

        xv = x_ref[:, :]
        dyv = dy_ref[:, :]
        mu = jnp.mean(xv, axis=1, keepdims=True)
        xc = xv - mu
        var = jnp.mean(xc * xc, axis=1, keepdims=True)
        xhat = xc * lax.rsqrt(var + 1e-5)
        dg = jnp.sum(dyv * xhat, axis=0)
        db = jnp.sum(dyv, axis=0)
        own_ref[:, :] = jnp.stack([dg, db])

        rdmas = []
        for k in range(1, N_DEV):
            j = N_DEV - k
            rd = pltpu.make_async_remote_copy(
                src_ref=own_ref,
                dst_ref=comm_ref.at[j],
                send_sem=send_sems.at[k],
                recv_sem=recv_sems.at[j],
                device_id=((my + k) % N_DEV,),
                device_id_type=pl.DeviceIdType.MESH,
            )
            rd.start()
            rdmas.append(rd)

        total = own_ref[:, :]
        for k, rd in zip(range(1, N_DEV), rdmas):
            rd.wait_send()
            rd.wait_recv()
            total = total + comm_ref[N_DEV - k]
        out_ref[:, :] = total

        for k in range(1, N_DEV):
            pl.semaphore_signal(
                exit_sems.at[N_DEV - k], inc=1,
                device_id=((my + k) % N_DEV,),
                device_id_type=pl.DeviceIdType.MESH,
            )
        for j in range(1, N_DEV):
            pl.semaphore_wait(exit_sems.at[j], 1)

    return pl.pallas_call(
        body,
        out_shape=jax.ShapeDtypeStruct((2, d), jnp.float32),
        in_specs=[
            pl.BlockSpec(memory_space=pltpu.VMEM),
            pl.BlockSpec(memory_space=pltpu.VMEM),
        ],
        out_specs=pl.BlockSpec(memory_space=pltpu.VMEM),
        scratch_shapes=[
            pltpu.VMEM((2, d), jnp.float32),
            pltpu.VMEM((N_DEV, 2, d), jnp.float32),
            pltpu.SemaphoreType.DMA((N_DEV,)),
            pltpu.SemaphoreType.DMA((N_DEV,)),
            pltpu.SemaphoreType.REGULAR((N_DEV,)),
        ],
    )(x.astype(jnp.float32), dy.astype(jnp.float32))


# device time: 16741 ns/iter; 1.1036x vs baseline; 1.1036x over previous
import jax
import jax.numpy as jnp
from jax import lax
from jax.experimental import pallas as pl
from jax.experimental.pallas import tpu as pltpu

N_DEV = 16


def kernel(x, dy, gamma):
    m, d = x.shape

    def body(x_ref, dy_ref, out_ref, own_ref, comm_ref,
             send_sems, recv_sems, exit_sems):
        my = lax.axis_index("i")

        xv = x_ref[:, :]
        dyv = dy_ref[:, :]
        mu = jnp.mean(xv, axis=1, keepdims=True)
        xc = xv - mu
        var = jnp.mean(xc * xc, axis=1, keepdims=True)
        xhat = xc * lax.rsqrt(var + 1e-5)
        dg = jnp.sum(dyv * xhat, axis=0)
        db = jnp.sum(dyv, axis=0)
        own_ref[:, :] = jnp.stack([dg, db])

        barrier_sem = pltpu.get_barrier_semaphore()
        for k in range(1, N_DEV):
            pl.semaphore_signal(
                barrier_sem, inc=1,
                device_id=((my + k) % N_DEV,),
                device_id_type=pl.DeviceIdType.MESH,
            )
        pl.semaphore_wait(barrier_sem, N_DEV - 1)

        rdmas = []
        for k in range(1, N_DEV):
            j = N_DEV - k
            rd = pltpu.make_async_remote_copy(
                src_ref=own_ref,
                dst_ref=comm_ref.at[j],
                send_sem=send_sems.at[k],
                recv_sem=recv_sems.at[j],
                device_id=((my + k) % N_DEV,),
                device_id_type=pl.DeviceIdType.MESH,
            )
            rd.start()
            rdmas.append(rd)

        total = own_ref[:, :]
        for k, rd in zip(range(1, N_DEV), rdmas):
            rd.wait_send()
            rd.wait_recv()
            total = total + comm_ref[N_DEV - k]
        out_ref[:, :] = total

        for k in range(1, N_DEV):
            pl.semaphore_signal(
                exit_sems.at[N_DEV - k], inc=1,
                device_id=((my + k) % N_DEV,),
                device_id_type=pl.DeviceIdType.MESH,
            )
        for j in range(1, N_DEV):
            pl.semaphore_wait(exit_sems.at[j], 1)

    return pl.pallas_call(
        body,
        out_shape=jax.ShapeDtypeStruct((2, d), jnp.float32),
        in_specs=[
            pl.BlockSpec(memory_space=pltpu.VMEM),
            pl.BlockSpec(memory_space=pltpu.VMEM),
        ],
        out_specs=pl.BlockSpec(memory_space=pltpu.VMEM),
        scratch_shapes=[
            pltpu.VMEM((2, d), jnp.float32),
            pltpu.VMEM((N_DEV, 2, d), jnp.float32),
            pltpu.SemaphoreType.DMA((N_DEV,)),
            pltpu.SemaphoreType.DMA((N_DEV,)),
            pltpu.SemaphoreType.REGULAR((N_DEV,)),
        ],
        compiler_params=pltpu.CompilerParams(collective_id=0),
    )(x.astype(jnp.float32), dy.astype(jnp.float32))


# device time: 14444 ns/iter; 1.2791x vs baseline; 1.1590x over previous
import jax
import jax.numpy as jnp
from jax import lax
from jax.experimental import pallas as pl
from jax.experimental.pallas import tpu as pltpu

N_DEV = 16


def kernel(x, dy, gamma):
    m, d = x.shape

    def body(x_ref, dy_ref, out_ref, own_ref, comm_ref,
             send_sems, recv_sems, exit_sems):
        my = lax.axis_index("i")

        barrier_sem = pltpu.get_barrier_semaphore()
        for k in range(1, N_DEV):
            pl.semaphore_signal(
                barrier_sem, inc=1,
                device_id=((my + k) % N_DEV,),
                device_id_type=pl.DeviceIdType.MESH,
            )

        xv = x_ref[:, :]
        dyv = dy_ref[:, :]
        mu = jnp.mean(xv, axis=1, keepdims=True)
        xc = xv - mu
        var = jnp.mean(xc * xc, axis=1, keepdims=True)
        xhat = xc * lax.rsqrt(var + 1e-5)
        dg = jnp.sum(dyv * xhat, axis=0)
        db = jnp.sum(dyv, axis=0)
        own_ref[:, :] = jnp.stack([dg, db])

        pl.semaphore_wait(barrier_sem, N_DEV - 1)

        rdmas = {}
        for k in range(1, N_DEV):
            j = N_DEV - k
            rd = pltpu.make_async_remote_copy(
                src_ref=own_ref,
                dst_ref=comm_ref.at[j],
                send_sem=send_sems.at[k],
                recv_sem=recv_sems.at[j],
                device_id=((my + k) % N_DEV,),
                device_id_type=pl.DeviceIdType.MESH,
            )
            rd.start()
            rdmas[k] = rd

        order = sorted(range(1, N_DEV), key=lambda k: min(k, N_DEV - k))
        total = own_ref[:, :]
        for k in order:
            rdmas[k].wait_recv()
            total = total + comm_ref[N_DEV - k]
            pl.semaphore_signal(
                exit_sems.at[N_DEV - k], inc=1,
                device_id=((my + k) % N_DEV,),
                device_id_type=pl.DeviceIdType.MESH,
            )
        out_ref[:, :] = total

        for k in range(1, N_DEV):
            rdmas[k].wait_send()

        for j in range(1, N_DEV):
            pl.semaphore_wait(exit_sems.at[j], 1)

    return pl.pallas_call(
        body,
        out_shape=jax.ShapeDtypeStruct((2, d), jnp.float32),
        in_specs=[
            pl.BlockSpec(memory_space=pltpu.VMEM),
            pl.BlockSpec(memory_space=pltpu.VMEM),
        ],
        out_specs=pl.BlockSpec(memory_space=pltpu.VMEM),
        scratch_shapes=[
            pltpu.VMEM((2, d), jnp.float32),
            pltpu.VMEM((N_DEV, 2, d), jnp.float32),
            pltpu.SemaphoreType.DMA((N_DEV,)),
            pltpu.SemaphoreType.DMA((N_DEV,)),
            pltpu.SemaphoreType.REGULAR((N_DEV,)),
        ],
        compiler_params=pltpu.CompilerParams(collective_id=0),
    )(x.astype(jnp.float32), dy.astype(jnp.float32))


# device time: 12378 ns/iter; 1.4926x vs baseline; 1.1669x over previous
import jax
import jax.numpy as jnp
from jax import lax
from jax.experimental import pallas as pl
from jax.experimental.pallas import tpu as pltpu

N_DEV = 16


def kernel(x, dy, gamma):
    m, d = x.shape

    def body(x_ref, dy_ref, out_ref, own_ref, psum_ref, pcomm, zcomm,
             psend, precv, zsend, zrecv, pready, zready):
        my = lax.axis_index("i")
        z = my // 4
        p = my % 4

        def plane_peer(a):
            return (4 * z + (p + a) % 4,)

        def z_peer(b):
            return (4 * ((z + b) % 4) + p,)

        for c in (1, 2, 3):
            pl.semaphore_signal(
                pready.at[(4 - c) % 4], inc=1,
                device_id=plane_peer(c),
                device_id_type=pl.DeviceIdType.MESH,
            )
            pl.semaphore_signal(
                zready.at[(4 - c) % 4], inc=1,
                device_id=z_peer(c),
                device_id_type=pl.DeviceIdType.MESH,
            )

        barrier_sem = pltpu.get_barrier_semaphore()
        for c in (1, 3):
            pl.semaphore_signal(
                barrier_sem, inc=1,
                device_id=plane_peer(c),
                device_id_type=pl.DeviceIdType.MESH,
            )

        xv = x_ref[:, :]
        dyv = dy_ref[:, :]
        mu = jnp.mean(xv, axis=1, keepdims=True)
        xc = xv - mu
        var = jnp.mean(xc * xc, axis=1, keepdims=True)
        xhat = xc * lax.rsqrt(var + 1e-5)
        dg = jnp.sum(dyv * xhat, axis=0)
        db = jnp.sum(dyv, axis=0)
        own_ref[:, :] = jnp.stack([dg, db])

        pl.semaphore_wait(barrier_sem, 2)

        prd = {}
        for a in (1, 2, 3):
            pl.semaphore_wait(pready.at[a], 1)
            rd = pltpu.make_async_remote_copy(
                src_ref=own_ref,
                dst_ref=pcomm.at[(4 - a) % 4],
                send_sem=psend.at[a],
                recv_sem=precv.at[(4 - a) % 4],
                device_id=plane_peer(a),
                device_id_type=pl.DeviceIdType.MESH,
            )
            rd.start()
            prd[a] = rd
        for a in (3, 1, 2):
            prd[a].wait_recv()
        psum = own_ref[:, :] + pcomm[1] + pcomm[2] + pcomm[3]
        psum_ref[:, :] = psum

        zrd = {}
        for b in (1, 2, 3):
            pl.semaphore_wait(zready.at[b], 1)
            rd = pltpu.make_async_remote_copy(
                src_ref=psum_ref,
                dst_ref=zcomm.at[(4 - b) % 4],
                send_sem=zsend.at[b],
                recv_sem=zrecv.at[(4 - b) % 4],
                device_id=z_peer(b),
                device_id_type=pl.DeviceIdType.MESH,
            )
            rd.start()
            zrd[b] = rd
        for b in (3, 2, 1):
            zrd[b].wait_recv()
        out_ref[:, :] = psum + zcomm[1] + zcomm[2] + zcomm[3]

        for a in (1, 2, 3):
            prd[a].wait_send()
            zrd[a].wait_send()

    return pl.pallas_call(
        body,
        out_shape=jax.ShapeDtypeStruct((2, d), jnp.float32),
        in_specs=[
            pl.BlockSpec(memory_space=pltpu.VMEM),
            pl.BlockSpec(memory_space=pltpu.VMEM),
        ],
        out_specs=pl.BlockSpec(memory_space=pltpu.VMEM),
        scratch_shapes=[
            pltpu.VMEM((2, d), jnp.float32),
            pltpu.VMEM((2, d), jnp.float32),
            pltpu.VMEM((4, 2, d), jnp.float32),
            pltpu.VMEM((4, 2, d), jnp.float32),
            pltpu.SemaphoreType.DMA((4,)),
            pltpu.SemaphoreType.DMA((4,)),
            pltpu.SemaphoreType.DMA((4,)),
            pltpu.SemaphoreType.DMA((4,)),
            pltpu.SemaphoreType.REGULAR((4,)),
            pltpu.SemaphoreType.REGULAR((4,)),
        ],
        compiler_params=pltpu.CompilerParams(collective_id=0),
    )(x.astype(jnp.float32), dy.astype(jnp.float32))
